# baseline (speedup 1.0000x reference)
import functools

import jax
import jax.numpy as jnp
from jax import lax
from jax.experimental import pallas as pl
from jax.experimental.pallas import tpu as pltpu
from jax.experimental.pallas import tpu_sc as plsc

NUM_EMB = 1000000
D = 32
B = 4096
L = 200
BTOT = B * L

_info = plsc.get_sparse_core_info()
NC, NS = _info.num_cores, _info.num_subcores
NW = NC * NS
LT = L // 8

CL = 2
CR = CL * 128
N_CH = L // CL
NBUF = 3

_mesh = plsc.VectorSubcoreMesh(core_axis_name="c", subcore_axis_name="s")

TCOLS = NUM_EMB // 128
WINL = 512
NWIN = NUM_EMB // WINL
RBUF = 3


@functools.partial(
    pl.kernel,
    mesh=_mesh,
    out_type=jax.ShapeDtypeStruct((NUM_EMB // 4, 128), jnp.float32),
    scratch_types=[
        pltpu.VMEM((RBUF, D, WINL), jnp.float32),
        pltpu.VMEM((RBUF, WINL // 4, 128), jnp.float32),
        [pltpu.SemaphoreType.DMA] * RBUF,
        [pltpu.SemaphoreType.DMA] * RBUF,
    ],
    compiler_params=pltpu.CompilerParams(needs_layout_passes=False),
)
def _relayout_kernel(tt_hbm, out_hbm, slab_v, t_v, g_sems, s_sems):
    w = lax.axis_index("s") * NC + lax.axis_index("c")
    iota = lax.iota(jnp.int32, 16)
    rot = [(iota + d) & 15 for d in range(16)]

    def start_read(win, b):
        pltpu.async_copy(
            tt_hbm.at[:, pl.ds(pl.multiple_of(win * WINL, 128), WINL)],
            slab_v.at[b],
            g_sems[b],
        )

    def wait_read(b):
        pltpu.make_async_copy(
            tt_hbm.at[:, pl.ds(0, WINL)], slab_v.at[b], g_sems[b]
        ).wait()

    def start_store(win, b):
        pltpu.async_copy(
            t_v.at[b], out_hbm.at[pl.ds(win * (WINL // 4), WINL // 4)],
            s_sems[b],
        )

    def wait_store(b):
        pltpu.make_async_copy(
            t_v.at[b], out_hbm.at[pl.ds(0, WINL // 4)], s_sems[b]
        ).wait()

    def transpose(b, m_blocks):
        sb = slab_v.at[b]
        tb = t_v.at[b]

        @plsc.parallel_loop(0, 2 * m_blocks, unroll=2)
        def tbody(i):
            m0 = (i & (m_blocks - 1)) * 16
            f0 = (i // m_blocks) * 16
            f_vec = iota + f0
            for d in range(16):
                m_vec = rot[d] + m0
                v = plsc.load_gather(sb, [f_vec, m_vec])
                plsc.store_scatter(
                    tb, [m_vec >> 2, ((m_vec & 3) << 5) + f_vec], v
                )

    for b in range(RBUF):
        start_read(w + b * NW, b)
    for b in range(RBUF):
        win = w + b * NW
        wait_read(b)
        transpose(b, WINL // 16)
        start_store(win, b)
        start_read(win + RBUF * NW, b)

    def body(o, carry):
        for b in range(RBUF):
            j = o * RBUF + b
            win = w + j * NW
            wait_read(b)
            wait_store(b)
            transpose(b, WINL // 16)
            start_store(win, b)
            start_read(win + RBUF * NW, b)
        return carry

    n_full = NWIN // NW
    n_extra = NWIN % NW
    lax.fori_loop(1, n_full // RBUF - 1, body, 0)

    for j in range((n_full // RBUF - 1) * RBUF, n_full):
        b = j % RBUF
        win = w + j * NW
        wait_read(b)
        wait_store(b)
        transpose(b, WINL // 16)
        start_store(win, b)
        if j + RBUF < n_full:
            start_read(win + RBUF * NW, b)
        elif j + RBUF == n_full:

            @pl.when(w < n_extra)
            def _():
                start_read(win + RBUF * NW, b)

    @pl.when(w < n_extra)
    def _extra():
        b = n_full % RBUF
        win = w + n_full * NW
        wait_read(b)
        wait_store(b)
        transpose(b, WINL // 16)
        start_store(win, b)

    for b in range(RBUF):
        wait_store(b)

    @pl.when(w == 0)
    def _tail():
        off = pl.multiple_of((TCOLS + 0 * w) * 128, 128)
        pltpu.sync_copy(
            tt_hbm.at[:, pl.ds(off, 128)], slab_v.at[0, :, pl.ds(0, 128)]
        )
        transpose(0, 4)
        pltpu.sync_copy(
            t_v.at[0, pl.ds(0, 16)], out_hbm.at[pl.ds(TCOLS * 32, 16)]
        )


@functools.partial(
    pl.kernel,
    mesh=_mesh,
    out_type=jax.ShapeDtypeStruct((L, D // 8, B // 128, 8, 128), jnp.float32),
    scratch_types=[
        pltpu.VMEM((LT, 1024), jnp.int32),
        pltpu.VMEM((NBUF, CR, D), jnp.float32),
        pltpu.VMEM((NBUF, CL, D // 8, 8, 128), jnp.float32),
        [pltpu.SemaphoreType.DMA] * NBUF,
        [pltpu.SemaphoreType.DMA] * NBUF,
    ],
    compiler_params=pltpu.CompilerParams(
        use_tc_tiling_on_sc=False, needs_layout_passes=False
    ),
)
def _gather_kernel(idx_hbm, table_hbm, out_hbm, idx_v, rows_v, t_v, g_sems, s_sems):
    w = lax.axis_index("s") * NC + lax.axis_index("c")

    for li in range(8):
        pltpu.sync_copy(idx_hbm.at[:, w, li], idx_v.at[:, pl.ds(li * 128, 128)])

    iota = lax.iota(jnp.int32, 16)
    rot = [(iota + d) & 15 for d in range(16)]

    def start_gather(c, b):
        l0 = c * CL
        lt = l0 // 8
        off = (l0 % 8) * 128
        pltpu.async_copy(
            table_hbm.at[idx_v.at[lt, pl.ds(off, CR)]], rows_v.at[b], g_sems[b]
        )

    def wait_gather(b):
        pltpu.make_async_copy(
            table_hbm.at[idx_v.at[0, pl.ds(0, CR)]], rows_v.at[b], g_sems[b]
        ).wait()

    def start_store(c, b):
        pltpu.async_copy(
            t_v.at[b], out_hbm.at[pl.ds(c * CL, CL), :, w], s_sems[b]
        )

    def wait_store(b):
        pltpu.make_async_copy(
            t_v.at[b], out_hbm.at[pl.ds(0, CL), :, 0], s_sems[b]
        ).wait()

    def transpose(b):
        rb = rows_v.at[b]
        tb = t_v.at[b]

        @plsc.parallel_loop(0, CL * 16, unroll=2)
        def tbody(i):
            lrel = i >> 4
            r = (i >> 1) & 7
            f0 = (i & 1) * 16
            lrel_vec = jnp.full((16,), 0, jnp.int32) + lrel
            bi_vec = iota + r * 16
            row_vec = bi_vec + lrel * 128
            for d in range(16):
                col = rot[d] + f0
                v = plsc.load_gather(rb, [row_vec, col])
                plsc.store_scatter(tb, [lrel_vec, col >> 3, col & 7, bi_vec], v)

    for b in range(NBUF):
        start_gather(b, b)
    for b in range(NBUF):
        wait_gather(b)
        transpose(b)
        start_store(b, b)
        start_gather(NBUF + b, b)

    def body(o, carry):
        for b in range(NBUF):
            c = o * NBUF + b
            wait_gather(b)
            wait_store(b)
            transpose(b)
            start_store(c, b)
            start_gather(c + NBUF, b)
        return carry

    lax.fori_loop(1, N_CH // NBUF - 1, body, 0)

    for c in range((N_CH // NBUF - 1) * NBUF, N_CH):
        b = c % NBUF
        wait_gather(b)
        wait_store(b)
        transpose(b)
        start_store(c, b)
        if c + NBUF < N_CH:
            start_gather(c + NBUF, b)
    for b in range(NBUF):
        wait_store(b)


def kernel(inputs, embedding):
    idx_view = inputs.T.reshape(LT, 8, B // 128, 128).transpose(0, 2, 1, 3)
    table_lin = _relayout_kernel(embedding.T).reshape(NUM_EMB, D)
    out5 = _gather_kernel(idx_view, table_lin)
    return out5.transpose(2, 4, 0, 1, 3).reshape(B, L, D)


# speedup vs baseline: 1.0125x; 1.0125x over previous
import functools

import jax
import jax.numpy as jnp
from jax import lax
from jax.experimental import pallas as pl
from jax.experimental.pallas import tpu as pltpu
from jax.experimental.pallas import tpu_sc as plsc

NUM_EMB = 1000000
D = 32
B = 4096
L = 200
BTOT = B * L

_info = plsc.get_sparse_core_info()
NC, NS = _info.num_cores, _info.num_subcores
NW = NC * NS
LT = L // 8

CL = 4
CR = CL * 128
N_CH = L // CL
NBUF = 2

_mesh = plsc.VectorSubcoreMesh(core_axis_name="c", subcore_axis_name="s")

TCOLS = NUM_EMB // 128
WINL = 512
NWIN = NUM_EMB // WINL
RBUF = 3


@functools.partial(
    pl.kernel,
    mesh=_mesh,
    out_type=jax.ShapeDtypeStruct((NUM_EMB // 4, 128), jnp.float32),
    scratch_types=[
        pltpu.VMEM((RBUF, D, WINL), jnp.float32),
        pltpu.VMEM((RBUF, WINL // 4, 128), jnp.float32),
        [pltpu.SemaphoreType.DMA] * RBUF,
        [pltpu.SemaphoreType.DMA] * RBUF,
    ],
    compiler_params=pltpu.CompilerParams(needs_layout_passes=False),
)
def _relayout_kernel(tt_hbm, out_hbm, slab_v, t_v, g_sems, s_sems):
    w = lax.axis_index("s") * NC + lax.axis_index("c")
    iota = lax.iota(jnp.int32, 16)
    rot = [(iota + d) & 15 for d in range(16)]

    def start_read(win, b):
        pltpu.async_copy(
            tt_hbm.at[:, pl.ds(pl.multiple_of(win * WINL, 128), WINL)],
            slab_v.at[b],
            g_sems[b],
        )

    def wait_read(b):
        pltpu.make_async_copy(
            tt_hbm.at[:, pl.ds(0, WINL)], slab_v.at[b], g_sems[b]
        ).wait()

    def start_store(win, b):
        pltpu.async_copy(
            t_v.at[b], out_hbm.at[pl.ds(win * (WINL // 4), WINL // 4)],
            s_sems[b],
        )

    def wait_store(b):
        pltpu.make_async_copy(
            t_v.at[b], out_hbm.at[pl.ds(0, WINL // 4)], s_sems[b]
        ).wait()

    def transpose(b, m_blocks):
        sb = slab_v.at[b]
        tb = t_v.at[b]

        @plsc.parallel_loop(0, 2 * m_blocks, unroll=2)
        def tbody(i):
            m0 = (i & (m_blocks - 1)) * 16
            f0 = (i // m_blocks) * 16
            f_vec = iota + f0
            for d in range(16):
                m_vec = rot[d] + m0
                v = plsc.load_gather(sb, [f_vec, m_vec])
                plsc.store_scatter(
                    tb, [m_vec >> 2, ((m_vec & 3) << 5) + f_vec], v
                )

    for b in range(RBUF):
        start_read(w + b * NW, b)
    for b in range(RBUF):
        win = w + b * NW
        wait_read(b)
        transpose(b, WINL // 16)
        start_store(win, b)
        start_read(win + RBUF * NW, b)

    def body(o, carry):
        for b in range(RBUF):
            j = o * RBUF + b
            win = w + j * NW
            wait_read(b)
            wait_store(b)
            transpose(b, WINL // 16)
            start_store(win, b)
            start_read(win + RBUF * NW, b)
        return carry

    n_full = NWIN // NW
    n_extra = NWIN % NW
    lax.fori_loop(1, n_full // RBUF - 1, body, 0)

    for j in range((n_full // RBUF - 1) * RBUF, n_full):
        b = j % RBUF
        win = w + j * NW
        wait_read(b)
        wait_store(b)
        transpose(b, WINL // 16)
        start_store(win, b)
        if j + RBUF < n_full:
            start_read(win + RBUF * NW, b)
        elif j + RBUF == n_full:

            @pl.when(w < n_extra)
            def _():
                start_read(win + RBUF * NW, b)

    @pl.when(w < n_extra)
    def _extra():
        b = n_full % RBUF
        win = w + n_full * NW
        wait_read(b)
        wait_store(b)
        transpose(b, WINL // 16)
        start_store(win, b)

    for b in range(RBUF):
        wait_store(b)

    @pl.when(w == 0)
    def _tail():
        off = pl.multiple_of((TCOLS + 0 * w) * 128, 128)
        pltpu.sync_copy(
            tt_hbm.at[:, pl.ds(off, 128)], slab_v.at[0, :, pl.ds(0, 128)]
        )
        transpose(0, 4)
        pltpu.sync_copy(
            t_v.at[0, pl.ds(0, 16)], out_hbm.at[pl.ds(TCOLS * 32, 16)]
        )


@functools.partial(
    pl.kernel,
    mesh=_mesh,
    out_type=jax.ShapeDtypeStruct((L, D // 8, B // 128, 8, 128), jnp.float32),
    scratch_types=[
        pltpu.VMEM((LT, 1024), jnp.int32),
        pltpu.VMEM((NBUF, CR, D), jnp.float32),
        pltpu.VMEM((NBUF, CL, D // 8, 8, 128), jnp.float32),
        [pltpu.SemaphoreType.DMA] * NBUF,
        [pltpu.SemaphoreType.DMA] * NBUF,
    ],
    compiler_params=pltpu.CompilerParams(
        use_tc_tiling_on_sc=False, needs_layout_passes=False
    ),
)
def _gather_kernel(idx_hbm, table_hbm, out_hbm, idx_v, rows_v, t_v, g_sems, s_sems):
    w = lax.axis_index("s") * NC + lax.axis_index("c")

    for li in range(8):
        pltpu.sync_copy(idx_hbm.at[:, w, li], idx_v.at[:, pl.ds(li * 128, 128)])

    iota = lax.iota(jnp.int32, 16)
    rot = [(iota + d) & 15 for d in range(16)]

    def start_gather(c, b):
        l0 = c * CL
        lt = l0 // 8
        off = (l0 % 8) * 128
        pltpu.async_copy(
            table_hbm.at[idx_v.at[lt, pl.ds(off, CR)]], rows_v.at[b], g_sems[b]
        )

    def wait_gather(b):
        pltpu.make_async_copy(
            table_hbm.at[idx_v.at[0, pl.ds(0, CR)]], rows_v.at[b], g_sems[b]
        ).wait()

    def start_store(c, b):
        pltpu.async_copy(
            t_v.at[b], out_hbm.at[pl.ds(c * CL, CL), :, w], s_sems[b]
        )

    def wait_store(b):
        pltpu.make_async_copy(
            t_v.at[b], out_hbm.at[pl.ds(0, CL), :, 0], s_sems[b]
        ).wait()

    def transpose(b):
        rb = rows_v.at[b]
        tb = t_v.at[b]

        @plsc.parallel_loop(0, CL * 16, unroll=2)
        def tbody(i):
            lrel = i >> 4
            r = (i >> 1) & 7
            f0 = (i & 1) * 16
            lrel_vec = jnp.full((16,), 0, jnp.int32) + lrel
            bi_vec = iota + r * 16
            row_vec = bi_vec + lrel * 128
            for d in range(16):
                col = rot[d] + f0
                v = plsc.load_gather(rb, [row_vec, col])
                plsc.store_scatter(tb, [lrel_vec, col >> 3, col & 7, bi_vec], v)

    for b in range(NBUF):
        start_gather(b, b)
    for b in range(NBUF):
        wait_gather(b)
        transpose(b)
        start_store(b, b)
        start_gather(NBUF + b, b)

    def body(o, carry):
        for b in range(NBUF):
            c = o * NBUF + b
            wait_gather(b)
            wait_store(b)
            transpose(b)
            start_store(c, b)
            start_gather(c + NBUF, b)
        return carry

    lax.fori_loop(1, N_CH // NBUF - 1, body, 0)

    for c in range((N_CH // NBUF - 1) * NBUF, N_CH):
        b = c % NBUF
        wait_gather(b)
        wait_store(b)
        transpose(b)
        start_store(c, b)
        if c + NBUF < N_CH:
            start_gather(c + NBUF, b)
    for b in range(NBUF):
        wait_store(b)


def kernel(inputs, embedding):
    idx_view = inputs.T.reshape(LT, 8, B // 128, 128).transpose(0, 2, 1, 3)
    table_lin = _relayout_kernel(embedding.T).reshape(NUM_EMB, D)
    out5 = _gather_kernel(idx_view, table_lin)
    return out5.transpose(2, 4, 0, 1, 3).reshape(B, L, D)
